# Initial kernel scaffold; baseline (speedup 1.0000x reference)
import jax
import jax.numpy as jnp
from jax.experimental import pallas as pl


def kernel(x, assignment_0, reversed_assignment_0, edge_index_0, edge_weight_0, edge_index_1, edge_weight_1, W_in, b_in, W_q, b_q, W_k, b_k, W_o, b_o):
    raise NotImplementedError("write your pallas kernel here")


# speedup vs baseline: 3.5780x; 3.5780x over previous
import functools
import jax
import jax.numpy as jnp
from jax.experimental import pallas as pl
from jax.experimental.pallas import tpu as pltpu

N, D, NH = 10000, 128, 1000


def _proj_body(x_ref, w_ref, b_ref, h_ref, hq_ref, hk_ref):
    acc = jax.lax.dot_general(x_ref[...], w_ref[...],
                              (((1,), (1,)), ((), ())),
                              preferred_element_type=jnp.float32)
    acc = acc + b_ref[...]
    h_ref[...] = acc[:, :D]
    hq_ref[...] = acc[:, D:2 * D]
    hk_ref[...] = acc[:, 2 * D:]


def _projections(x2, wcat, bcat):
    blk = 1000
    grid = (N // blk,)
    out_sds = jax.ShapeDtypeStruct((N, D), jnp.float32)
    return pl.pallas_call(
        _proj_body,
        grid=grid,
        in_specs=[
            pl.BlockSpec((blk, D), lambda i: (i, 0)),
            pl.BlockSpec((3 * D, D), lambda i: (0, 0)),
            pl.BlockSpec((1, 3 * D), lambda i: (0, 0)),
        ],
        out_specs=[pl.BlockSpec((blk, D), lambda i: (i, 0))] * 3,
        out_shape=[out_sds, out_sds, out_sds],
    )(x2, wcat, bcat)


def _gnn_layer_jnp(x, h_q, h_k, edge_index, edge_attr, W_o, b_o):
    n = x.shape[0]
    src = edge_index[0]
    dst = edge_index[1]
    sim = jnp.einsum('ed,ed->e', h_q[src], h_k[dst]) / jnp.sqrt(jnp.float32(D))
    bea = edge_attr * sim
    w = jax.nn.softmax(bea)
    msg = w[:, None] * x[src]
    aggr = jnp.zeros((n, D), dtype=x.dtype).at[dst].add(msg)
    return jax.nn.relu(aggr @ W_o.T + b_o)


def kernel(x, assignment_0, reversed_assignment_0, edge_index_0, edge_weight_0,
           edge_index_1, edge_weight_1, W_in, b_in, W_q, b_q, W_k, b_k, W_o, b_o):
    x2 = x[0]
    wcat = jnp.concatenate([W_in, W_q, W_k], axis=0)
    bcat = jnp.concatenate([b_in, b_q, b_k])[None, :]
    h, h_q, h_k = _projections(x2, wcat, bcat)
    h_prime = x2 + h
    h_prime = h_prime + _gnn_layer_jnp(h, h_q, h_k, edge_index_0, edge_weight_0, W_o, b_o)
    H = assignment_0.T @ h
    HQ = assignment_0.T @ h_q
    HK = assignment_0.T @ h_k
    h_hat = _gnn_layer_jnp(H, HQ, HK, edge_index_1, edge_weight_1, W_o, b_o)
    out = h_prime + reversed_assignment_0 @ h_hat
    return out[None]


# speedup vs baseline: 9.8042x; 2.7401x over previous
import functools

import jax
import jax.numpy as jnp
from jax import lax
from jax.experimental import pallas as pl
from jax.experimental.pallas import tpu as pltpu
from jax.experimental.pallas import tpu_sc as plsc

N, D, NH = 10000, 128, 1000
E0, E1 = 160000, 16000
NC, NS, NW, L = 2, 16, 32, 16
C = 64
HD = D // 2
NP0 = 10112
NP1 = 1024
INV_SQRT_D = 1.0 / (128.0 ** 0.5)


def _proj_body(x_ref, w_ref, b_ref, h_ref, hq_ref, hk_ref):
    acc = lax.dot_general(x_ref[...], w_ref[...], (((1,), (1,)), ((), ())),
                          preferred_element_type=jnp.float32)
    acc = acc + b_ref[...]
    h_ref[...] = acc[:, :D]
    hq_ref[...] = acc[:, D:2 * D]
    hk_ref[...] = acc[:, 2 * D:]


def _projections(x2, wcat, bcat):
    blk = 1000
    full = jax.ShapeDtypeStruct((N, D), jnp.float32)
    return pl.pallas_call(
        _proj_body,
        grid=(N // blk,),
        in_specs=[
            pl.BlockSpec((blk, D), lambda i: (i, 0)),
            pl.BlockSpec((3 * D, D), lambda i: (0, 0)),
            pl.BlockSpec((1, 3 * D), lambda i: (0, 0)),
        ],
        out_specs=[pl.BlockSpec((blk, D), lambda i: (i, 0))] * 3,
        out_shape=[full, full, full],
    )(x2, wcat, bcat)


def _make_score_kernel(nch, e_real):
    epw = nch * C
    mesh = plsc.VectorSubcoreMesh(core_axis_name="c", subcore_axis_name="s")

    @functools.partial(
        pl.kernel,
        out_type=[
            jax.ShapeDtypeStruct((NW * epw,), jnp.float32),
            jax.ShapeDtypeStruct((NW, L), jnp.float32),
        ],
        mesh=mesh,
        scratch_types=[
            pltpu.VMEM((nch, C), jnp.int32),
            pltpu.VMEM((nch, C), jnp.int32),
            pltpu.VMEM((epw,), jnp.float32),
            pltpu.VMEM((epw,), jnp.float32),
            pltpu.VMEM((C, D), jnp.float32),
            pltpu.VMEM((C, D), jnp.float32),
            pltpu.VMEM((C, D), jnp.float32),
            pltpu.VMEM((C, D), jnp.float32),
            pltpu.VMEM((L,), jnp.float32),
            pltpu.SemaphoreType.DMA,
            pltpu.SemaphoreType.DMA,
            pltpu.SemaphoreType.DMA,
            pltpu.SemaphoreType.DMA,
        ],
    )
    def score_kernel(hq_hbm, hk_hbm, src_hbm, dst_hbm, ew_hbm,
                     u_out, z_out,
                     srcw, dstw, eww, uw, qb0, qb1, kb0, kb1, zbuf,
                     sq0, sq1, sk0, sk1):
        cid = lax.axis_index("c")
        sid = lax.axis_index("s")
        wid = sid * NC + cid
        gbase = wid * epw

        qb = (qb0, qb1)
        kb = (kb0, kb1)
        sq = (sq0, sq1)
        sk = (sk0, sk1)

        ione = jax.lax.iota(jnp.int32, L)
        zeros16 = jnp.zeros((L,), jnp.float32)

        def lanesum(v):
            for k in (8, 4, 2, 1):
                p = jnp.bitwise_xor(ione, k)
                v = v + v.at[p].get(mode="promise_in_bounds")
            return v

        pltpu.sync_copy(src_hbm.at[pl.ds(wid * nch, nch)], srcw)
        pltpu.sync_copy(dst_hbm.at[pl.ds(wid * nch, nch)], dstw)
        pltpu.sync_copy(ew_hbm.at[pl.ds(gbase, epw)], eww)

        def issue(ch, b):
            pltpu.async_copy(hq_hbm.at[srcw.at[ch]], qb[b], sq[b])
            pltpu.async_copy(hk_hbm.at[dstw.at[ch]], kb[b], sk[b])

        def wait(b):
            pltpu.make_async_copy(hq_hbm.at[pl.ds(0, C)], qb[b], sq[b]).wait()
            pltpu.make_async_copy(hk_hbm.at[pl.ds(0, C)], kb[b], sk[b]).wait()

        def process(ch, b, zacc):
            qr, kr = qb[b], kb[b]
            for g in range(C // L):
                sims = zeros16
                for e in range(L):
                    le = g * L + e
                    acc = qr[le, pl.ds(0, L)] * kr[le, pl.ds(0, L)]
                    for j in range(1, D // L):
                        acc = acc + qr[le, pl.ds(j * L, L)] * kr[le, pl.ds(j * L, L)]
                    sims = jnp.where(ione == e, lanesum(acc), sims)
                ews = eww[pl.ds(ch * C + g * L, L)]
                bea = ews * sims * INV_SQRT_D
                in_range = (gbase + ch * C + g * L + ione) < e_real
                u = jnp.where(in_range, jnp.exp(bea), 0.0)
                zacc = zacc + u
                uw[pl.ds(ch * C + g * L, L)] = u
            return zacc

        issue(0, 0)
        issue(1, 1)

        def pair_body(p, zacc):
            for b in range(2):
                ch = 2 * p + b
                wait(b)
                zacc = process(ch, b, zacc)

                @pl.when(ch + 2 < nch)
                def _():
                    issue(ch + 2, b)
            return zacc

        zacc = lax.fori_loop(0, nch // 2, pair_body, zeros16)

        zbuf[...] = zacc
        pltpu.sync_copy(zbuf, z_out.at[wid])
        pltpu.sync_copy(uw, u_out.at[pl.ds(gbase, epw)])

    return score_kernel


def _make_scatter_kernel(nch):
    epw = nch * C
    rpt = NP0 // NS
    mesh = plsc.VectorSubcoreMesh(core_axis_name="c", subcore_axis_name="s")

    @functools.partial(
        pl.kernel,
        out_type=jax.ShapeDtypeStruct((NC, NP0, D), jnp.float32),
        mesh=mesh,
        scratch_types=[
            pltpu.VMEM((nch, C), jnp.int32),
            pltpu.VMEM((nch, C), jnp.int32),
            pltpu.VMEM((C,), jnp.float32),
            pltpu.VMEM((C,), jnp.float32),
            pltpu.VMEM((C, D), jnp.float32),
            pltpu.VMEM((C, D), jnp.float32),
            pltpu.VMEM((C, D), jnp.float32),
            pltpu.VMEM_SHARED((NP0, D), jnp.float32),
            pltpu.SemaphoreType.DMA,
            pltpu.SemaphoreType.DMA,
            pltpu.SemaphoreType.DMA,
            pltpu.SemaphoreType.DMA,
        ],
    )
    def scatter_kernel(h_hbm, src_hbm, dst_hbm, u_hbm,
                       aggr_out,
                       srcw, dstw, ub0, ub1, hb0, hb1, msgbuf, aggr_sh,
                       sh0, sh1, su0, su1):
        cid = lax.axis_index("c")
        sid = lax.axis_index("s")
        wid = sid * NC + cid

        hb = (hb0, hb1)
        ub = (ub0, ub1)
        sh = (sh0, sh1)
        su = (su0, su1)
        zeros16 = jnp.zeros((L,), jnp.float32)

        pltpu.sync_copy(src_hbm.at[pl.ds(wid * nch, nch)], srcw)
        pltpu.sync_copy(dst_hbm.at[pl.ds(wid * nch, nch)], dstw)

        for r in range(C):
            for j in range(D // L):
                msgbuf[r, pl.ds(j * L, L)] = zeros16
        for off in range(0, rpt, C):
            nrows = min(C, rpt - off)
            pltpu.sync_copy(msgbuf.at[pl.ds(0, nrows)],
                            aggr_sh.at[pl.ds(sid * rpt + off, nrows)])
        plsc.subcore_barrier()

        def issue(ch, b):
            pltpu.async_copy(h_hbm.at[srcw.at[ch]], hb[b], sh[b])
            pltpu.async_copy(u_hbm.at[pl.ds(wid * epw + ch * C, C)], ub[b], su[b])

        def wait(b):
            pltpu.make_async_copy(h_hbm.at[pl.ds(0, C)], hb[b], sh[b]).wait()
            pltpu.make_async_copy(u_hbm.at[pl.ds(0, C)], ub[b], su[b]).wait()

        def process(ch, b):
            hr = hb[b]
            for g in range(C // L):
                ug = ub[b][pl.ds(g * L, L)]
                for e in range(L):
                    le = g * L + e
                    us = ug[e]
                    for j in range(D // L):
                        msgbuf[le, pl.ds(j * L, L)] = hr[le, pl.ds(j * L, L)] * us
            pltpu.sync_copy(msgbuf, aggr_sh.at[dstw.at[ch]], add=True)

        issue(0, 0)
        issue(1, 1)

        def pair_body(p, carry):
            for b in range(2):
                ch = 2 * p + b
                wait(b)
                process(ch, b)

                @pl.when(ch + 2 < nch)
                def _():
                    issue(ch + 2, b)
            return carry

        lax.fori_loop(0, nch // 2, pair_body, 0)

        plsc.subcore_barrier()
        pltpu.sync_copy(aggr_sh.at[pl.ds(sid * rpt, rpt)],
                        aggr_out.at[cid, pl.ds(sid * rpt, rpt)])

    return scatter_kernel


def _make_edge_kernel(nch, e_real):
    epw = nch * C
    rpt = NP1 // NS
    mesh = plsc.VectorSubcoreMesh(core_axis_name="c", subcore_axis_name="s")

    @functools.partial(
        pl.kernel,
        out_type=[
            jax.ShapeDtypeStruct((NC, NP1, D), jnp.float32),
            jax.ShapeDtypeStruct((NW, L), jnp.float32),
        ],
        mesh=mesh,
        scratch_types=[
            pltpu.VMEM((nch, C), jnp.int32),
            pltpu.VMEM((nch, C), jnp.int32),
            pltpu.VMEM((epw,), jnp.float32),
            pltpu.VMEM((C, D), jnp.float32),
            pltpu.VMEM((C, D), jnp.float32),
            pltpu.VMEM((C, D), jnp.float32),
            pltpu.VMEM((C, D), jnp.float32),
            pltpu.VMEM((C, D), jnp.float32),
            pltpu.VMEM((C, D), jnp.float32),
            pltpu.VMEM((C, D), jnp.float32),
            pltpu.VMEM((L,), jnp.float32),
            pltpu.VMEM_SHARED((NP1, D), jnp.float32),
            pltpu.SemaphoreType.DMA,
            pltpu.SemaphoreType.DMA,
            pltpu.SemaphoreType.DMA,
            pltpu.SemaphoreType.DMA,
            pltpu.SemaphoreType.DMA,
            pltpu.SemaphoreType.DMA,
        ],
    )
    def edge_kernel(h_hbm, hq_hbm, hk_hbm, src_hbm, dst_hbm, ew_hbm,
                    aggr_out, z_out,
                    srcw, dstw, eww, qb0, qb1, kb0, kb1, hb0, hb1,
                    msgbuf, zbuf, aggr_sh,
                    sq0, sq1, sk0, sk1, sh0, sh1):
        cid = lax.axis_index("c")
        sid = lax.axis_index("s")
        wid = sid * NC + cid
        gbase = wid * epw

        qb = (qb0, qb1)
        kb = (kb0, kb1)
        hb = (hb0, hb1)
        sq = (sq0, sq1)
        sk = (sk0, sk1)
        sh = (sh0, sh1)

        ione = jax.lax.iota(jnp.int32, L)
        zeros16 = jnp.zeros((L,), jnp.float32)

        def lanesum(v):
            for k in (8, 4, 2, 1):
                p = jnp.bitwise_xor(ione, k)
                v = v + v.at[p].get(mode="promise_in_bounds")
            return v

        pltpu.sync_copy(src_hbm.at[pl.ds(wid * nch, nch)], srcw)
        pltpu.sync_copy(dst_hbm.at[pl.ds(wid * nch, nch)], dstw)
        pltpu.sync_copy(ew_hbm.at[pl.ds(gbase, epw)], eww)

        for r in range(C):
            for j in range(D // L):
                msgbuf[r, pl.ds(j * L, L)] = zeros16
        pltpu.sync_copy(msgbuf, aggr_sh.at[pl.ds(sid * rpt, rpt)])
        plsc.subcore_barrier()

        def issue(ch, b):
            idx_s = srcw.at[ch]
            pltpu.async_copy(hq_hbm.at[idx_s], qb[b], sq[b])
            pltpu.async_copy(hk_hbm.at[dstw.at[ch]], kb[b], sk[b])
            pltpu.async_copy(h_hbm.at[idx_s], hb[b], sh[b])

        def wait(b):
            pltpu.make_async_copy(hq_hbm.at[pl.ds(0, C)], qb[b], sq[b]).wait()
            pltpu.make_async_copy(hk_hbm.at[pl.ds(0, C)], kb[b], sk[b]).wait()
            pltpu.make_async_copy(h_hbm.at[pl.ds(0, C)], hb[b], sh[b]).wait()

        def process(ch, b, zacc):
            qr, kr, hr = qb[b], kb[b], hb[b]
            for g in range(C // L):
                sims = zeros16
                for e in range(L):
                    le = g * L + e
                    acc = qr[le, pl.ds(0, L)] * kr[le, pl.ds(0, L)]
                    for j in range(1, D // L):
                        acc = acc + qr[le, pl.ds(j * L, L)] * kr[le, pl.ds(j * L, L)]
                    sims = jnp.where(ione == e, lanesum(acc), sims)
                ews = eww[pl.ds(ch * C + g * L, L)]
                bea = ews * sims * INV_SQRT_D
                in_range = (gbase + ch * C + g * L + ione) < e_real
                u = jnp.where(in_range, jnp.exp(bea), 0.0)
                zacc = zacc + u
                for e in range(L):
                    le = g * L + e
                    us = u[e]
                    for j in range(D // L):
                        msgbuf[le, pl.ds(j * L, L)] = hr[le, pl.ds(j * L, L)] * us
            pltpu.sync_copy(msgbuf, aggr_sh.at[dstw.at[ch]], add=True)
            return zacc

        issue(0, 0)
        issue(1, 1)

        def pair_body(p, zacc):
            for b in range(2):
                ch = 2 * p + b
                wait(b)
                zacc = process(ch, b, zacc)

                @pl.when(ch + 2 < nch)
                def _():
                    issue(ch + 2, b)
            return zacc

        zacc = lax.fori_loop(0, nch // 2, pair_body, zeros16)

        zbuf[...] = zacc
        pltpu.sync_copy(zbuf, z_out.at[wid])
        plsc.subcore_barrier()
        pltpu.sync_copy(aggr_sh.at[pl.ds(sid * rpt, rpt)],
                        aggr_out.at[cid, pl.ds(sid * rpt, rpt)])

    return edge_kernel


def _combine0_body(parts_ref, z_ref, x_ref, h_ref, w_ref, b_ref, out_ref):
    z = jnp.sum(z_ref[...])
    agg = (parts_ref[0] + parts_ref[1]) * (1.0 / z)
    y = lax.dot_general(agg, w_ref[...], (((1,), (1,)), ((), ())),
                        preferred_element_type=jnp.float32) + b_ref[...]
    out_ref[...] = jnp.maximum(y, 0.0) + x_ref[...] + h_ref[...]


def _combine0(parts, zpart, x2, h, W_o, b_o2):
    blk = 1000
    return pl.pallas_call(
        _combine0_body,
        grid=(N // blk,),
        in_specs=[
            pl.BlockSpec((2, blk, D), lambda i: (0, i, 0)),
            pl.BlockSpec((NW, L), lambda i: (0, 0)),
            pl.BlockSpec((blk, D), lambda i: (i, 0)),
            pl.BlockSpec((blk, D), lambda i: (i, 0)),
            pl.BlockSpec((D, D), lambda i: (0, 0)),
            pl.BlockSpec((1, D), lambda i: (0, 0)),
        ],
        out_specs=pl.BlockSpec((blk, D), lambda i: (i, 0)),
        out_shape=jax.ShapeDtypeStruct((N, D), jnp.float32),
    )(parts, zpart, x2, h, W_o, b_o2)


def _pool_body(a_ref, h_ref, hq_ref, hk_ref, H_ref, HQ_ref, HK_ref):
    i = pl.program_id(0)
    a = a_ref[...]

    def mm(v_ref):
        return lax.dot_general(a, v_ref[...], (((0,), (0,)), ((), ())),
                               preferred_element_type=jnp.float32)

    @pl.when(i == 0)
    def _():
        H_ref[...] = mm(h_ref)
        HQ_ref[...] = mm(hq_ref)
        HK_ref[...] = mm(hk_ref)

    @pl.when(i > 0)
    def _():
        H_ref[...] += mm(h_ref)
        HQ_ref[...] += mm(hq_ref)
        HK_ref[...] += mm(hk_ref)


def _pool(assign, h, hq, hk):
    blk = 1000
    out_sds = jax.ShapeDtypeStruct((NH, D), jnp.float32)
    return pl.pallas_call(
        _pool_body,
        grid=(N // blk,),
        in_specs=[
            pl.BlockSpec((blk, NH), lambda i: (i, 0)),
            pl.BlockSpec((blk, D), lambda i: (i, 0)),
            pl.BlockSpec((blk, D), lambda i: (i, 0)),
            pl.BlockSpec((blk, D), lambda i: (i, 0)),
        ],
        out_specs=[pl.BlockSpec((NH, D), lambda i: (0, 0))] * 3,
        out_shape=[out_sds, out_sds, out_sds],
    )(assign, h, hq, hk)


def _unpool_body(parts_ref, z_ref, r_ref, hp_ref, w_ref, b_ref, out_ref):
    z = jnp.sum(z_ref[...])
    agg = (parts_ref[0] + parts_ref[1]) * (1.0 / z)
    h_hat = lax.dot_general(agg, w_ref[...], (((1,), (1,)), ((), ())),
                            preferred_element_type=jnp.float32) + b_ref[...]
    h_hat = jnp.maximum(h_hat, 0.0)
    up = lax.dot_general(r_ref[...], h_hat, (((1,), (0,)), ((), ())),
                         preferred_element_type=jnp.float32)
    out_ref[...] = hp_ref[...] + up


def _unpool(parts1, z1part, rev_assign, h_prime, W_o, b_o2):
    blk = 1000
    return pl.pallas_call(
        _unpool_body,
        grid=(N // blk,),
        in_specs=[
            pl.BlockSpec((2, NH, D), lambda i: (0, 0, 0)),
            pl.BlockSpec((NW, L), lambda i: (0, 0)),
            pl.BlockSpec((blk, NH), lambda i: (i, 0)),
            pl.BlockSpec((blk, D), lambda i: (i, 0)),
            pl.BlockSpec((D, D), lambda i: (0, 0)),
            pl.BlockSpec((1, D), lambda i: (0, 0)),
        ],
        out_specs=pl.BlockSpec((blk, D), lambda i: (i, 0)),
        out_shape=jax.ShapeDtypeStruct((N, D), jnp.float32),
    )(parts1, z1part, rev_assign, h_prime, W_o, b_o2)


def _pad_edges(edge_index, edge_weight, e_real, nch):
    total = NW * nch * C
    pad = total - e_real
    src = jnp.pad(edge_index[0], (0, pad)).reshape(NW * nch, C)
    dst = jnp.pad(edge_index[1], (0, pad)).reshape(NW * nch, C)
    ew = jnp.pad(edge_weight, (0, pad)).reshape(NW * nch * C)
    return src, dst, ew


def kernel(x, assignment_0, reversed_assignment_0, edge_index_0, edge_weight_0,
           edge_index_1, edge_weight_1, W_in, b_in, W_q, b_q, W_k, b_k, W_o, b_o):
    x2 = x[0]
    wcat = jnp.concatenate([W_in, W_q, W_k], axis=0)
    bcat = jnp.concatenate([b_in, b_q, b_k])[None, :]
    b_o2 = b_o[None, :]

    h, h_q, h_k = _projections(x2, wcat, bcat)

    nch0 = (E0 + NW * C - 1) // (NW * C)
    nch0 += nch0 % 2
    src0, dst0, ew0 = _pad_edges(edge_index_0, edge_weight_0, E0, nch0)
    u0, zpart0 = _make_score_kernel(nch0, E0)(h_q, h_k, src0, dst0, ew0)
    parts0 = _make_scatter_kernel(nch0)(h, src0, dst0, u0)
    h_prime = _combine0(parts0[:, :N, :], zpart0, x2, h, W_o, b_o2)

    H, HQ, HK = _pool(assignment_0, h, h_q, h_k)

    nch1 = (E1 + NW * C - 1) // (NW * C)
    nch1 += nch1 % 2
    src1, dst1, ew1 = _pad_edges(edge_index_1, edge_weight_1, E1, nch1)
    parts1, zpart1 = _make_edge_kernel(nch1, E1)(H, HQ, HK, src1, dst1, ew1)

    out = _unpool(parts1[:, :NH, :], zpart1, reversed_assignment_0, h_prime, W_o, b_o2)
    return out[None]
